# Initial kernel scaffold; baseline (speedup 1.0000x reference)
import jax
import jax.numpy as jnp
from jax.experimental import pallas as pl


def kernel(x, tables, W1, b1, W2, b2):
    raise NotImplementedError("write your pallas kernel here")


# speedup vs baseline: 2.7482x; 2.7482x over previous
import functools

import numpy as np
import jax
import jax.numpy as jnp
from jax import lax
from jax.experimental import pallas as pl
from jax.experimental.pallas import tpu as pltpu
from jax.experimental.pallas import tpu_sc as plsc

_L = 8
_FPL = 2
_T = 2 ** 19
_N_MIN = 32
_B_GROWTH = float(np.exp(np.log(2048.0 / _N_MIN) / (_L - 1)))
_RES = [int(np.floor(_N_MIN * (_B_GROWTH ** l))) for l in range(_L)]
_TS = [min(_T, (r + 1) ** 3) for r in _RES]
_DENSE = [(r + 1) ** 3 <= _T for r in _RES]
_K2 = -1640531535
_K3 = 805459861
_MASK = _T - 1
_TSP = [2 * s + ((-2 * s) % 128) for s in _TS]

_NC, _NS, _LANES = 2, 16, 16
_NW = _NC * _NS

_N = 262144
_PPW = _N // _NW
_C = 1024
_NCHUNK = _PPW // _C
_NVEC = _C // _LANES

_CORNERS = [(dz, dy, dx) for dz in (0, 1) for dy in (0, 1) for dx in (0, 1)]


def _enc_body(xT, *rest):
    tabs = rest[:_L]
    enc = rest[_L]
    (xv0, xv1, xv2, wxs, wys, wzs, idxE, idxO, bufE, bufO, outE,
     outO) = rest[_L + 1:]

    cid = lax.axis_index("c")
    sid = lax.axis_index("s")
    wid = sid * _NC + cid
    base = wid * _PPW

    pltpu.sync_copy(xT.at[pl.ds(0 * _N + base, _PPW)], xv0)
    pltpu.sync_copy(xT.at[pl.ds(1 * _N + base, _PPW)], xv1)
    pltpu.sync_copy(xT.at[pl.ds(2 * _N + base, _PPW)], xv2)

    for l in range(_L):
        res = _RES[l]
        resf = float(res)

        def chunk_body(ch, carry, l=l, res=res, resf=resf):
            cbase = pl.multiple_of(ch * _C, _C)

            def pass1(i, carry2):
                off = pl.multiple_of(cbase + i * _LANES, _LANES)
                loc = pl.multiple_of(i * _LANES, _LANES)
                qs = []
                for xv, ws in ((xv0, wxs), (xv1, wys), (xv2, wzs)):
                    xd = xv[pl.ds(off, _LANES)]
                    xn = xd * 0.5 + 0.5
                    pos = xn * resf
                    q = pos.astype(jnp.int32)
                    w = pos - q.astype(jnp.float32)
                    ws[pl.ds(loc, _LANES)] = w
                    qs.append(q)
                q0, q1, q2 = qs
                if _DENSE[l]:
                    s = res + 1
                    eb = (q0 + q1 * s + q2 * (s * s)) * 2
                    for c, (dz, dy, dx) in enumerate(_CORNERS):
                        cst = 2 * (dx + dy * s + dz * s * s)
                        dst = pl.multiple_of(c * _C + i * _LANES, _LANES)
                        e = eb + cst
                        idxE[pl.ds(dst, _LANES)] = e
                        idxO[pl.ds(dst, _LANES)] = e + 1
                else:
                    ay0 = q1 * _K2
                    az0 = q2 * _K3
                    ay1 = ay0 + _K2
                    az1 = az0 + _K3
                    ax1 = q0 + 1
                    for c, (dz, dy, dx) in enumerate(_CORNERS):
                        hyz = (ay1 if dy else ay0) ^ (az1 if dz else az0)
                        h = (ax1 if dx else q0) ^ hyz
                        e = (h & _MASK) * 2
                        dst = pl.multiple_of(c * _C + i * _LANES, _LANES)
                        idxE[pl.ds(dst, _LANES)] = e
                        idxO[pl.ds(dst, _LANES)] = e + 1
                return carry2

            lax.fori_loop(0, _NVEC, pass1, 0)

            pltpu.sync_copy(tabs[l].at[idxE], bufE)
            pltpu.sync_copy(tabs[l].at[idxO], bufO)

            def pass2(i, carry2):
                loc = pl.multiple_of(i * _LANES, _LANES)
                off = pl.multiple_of(cbase + i * _LANES, _LANES)
                wx = wxs[pl.ds(loc, _LANES)]
                wy = wys[pl.ds(loc, _LANES)]
                wz = wzs[pl.ds(loc, _LANES)]
                ux = 1.0 - wx
                uy = 1.0 - wy
                uz = 1.0 - wz
                tyz = {(0, 0): uy * uz, (0, 1): wy * uz,
                       (1, 0): uy * wz, (1, 1): wy * wz}
                acc0 = jnp.zeros((_LANES,), jnp.float32)
                acc1 = jnp.zeros((_LANES,), jnp.float32)
                for c, (dz, dy, dx) in enumerate(_CORNERS):
                    wfac = tyz[(dy, dz)] * (wx if dx else ux)
                    src = pl.multiple_of(c * _C + i * _LANES, _LANES)
                    acc0 = acc0 + wfac * bufE[pl.ds(src, _LANES)]
                    acc1 = acc1 + wfac * bufO[pl.ds(src, _LANES)]
                outE[pl.ds(off, _LANES)] = acc0
                outO[pl.ds(off, _LANES)] = acc1
                return carry2

            lax.fori_loop(0, _NVEC, pass2, 0)
            return carry

        lax.fori_loop(0, _NCHUNK, chunk_body, 0)

        pltpu.sync_copy(outE, enc.at[pl.ds((2 * l) * _N + base, _PPW)])
        pltpu.sync_copy(outO, enc.at[pl.ds((2 * l + 1) * _N + base, _PPW)])


def _encode_sc(xT, flat_tables):
    mesh = plsc.VectorSubcoreMesh(core_axis_name="c", subcore_axis_name="s")
    scratch = [
        pltpu.VMEM((_PPW,), jnp.float32),
        pltpu.VMEM((_PPW,), jnp.float32),
        pltpu.VMEM((_PPW,), jnp.float32),
        pltpu.VMEM((_C,), jnp.float32),
        pltpu.VMEM((_C,), jnp.float32),
        pltpu.VMEM((_C,), jnp.float32),
        pltpu.VMEM((8 * _C,), jnp.int32),
        pltpu.VMEM((8 * _C,), jnp.int32),
        pltpu.VMEM((8 * _C,), jnp.float32),
        pltpu.VMEM((8 * _C,), jnp.float32),
        pltpu.VMEM((_PPW,), jnp.float32),
        pltpu.VMEM((_PPW,), jnp.float32),
    ]
    fn = pl.kernel(
        _enc_body,
        out_type=jax.ShapeDtypeStruct((2 * _L * _N,), jnp.float32),
        mesh=mesh,
        scratch_types=scratch,
    )
    return fn(xT, *flat_tables)


def _mlp_body(enc_ref, w1_ref, b1_ref, w2_ref, b2_ref, o_ref):
    e = enc_ref[...]
    w1 = w1_ref[...]
    h = lax.dot_general(w1, e, (((0,), (0,)), ((), ())),
                        preferred_element_type=jnp.float32)
    h = jnp.maximum(h + b1_ref[...], 0.0)
    w2 = w2_ref[...]
    o = lax.dot_general(w2, h, (((0,), (0,)), ((), ())),
                        preferred_element_type=jnp.float32)
    o_ref[...] = jax.nn.sigmoid(o + b2_ref[...])


def _mlp_tc(encT, W1, b1, W2, b2):
    bn = 8192
    n = encT.shape[1]
    grid = (n // bn,)
    return pl.pallas_call(
        _mlp_body,
        grid=grid,
        in_specs=[
            pl.BlockSpec((2 * _L, bn), lambda j: (0, j)),
            pl.BlockSpec((2 * _L, 32), lambda j: (0, 0)),
            pl.BlockSpec((32, 1), lambda j: (0, 0)),
            pl.BlockSpec((32, 1), lambda j: (0, 0)),
            pl.BlockSpec((1, 1), lambda j: (0, 0)),
        ],
        out_specs=pl.BlockSpec((1, bn), lambda j: (0, j)),
        out_shape=jax.ShapeDtypeStruct((1, n), jnp.float32),
    )(encT, W1, b1.reshape(32, 1), W2, b2.reshape(1, 1))


def kernel(x, tables, W1, b1, W2, b2):
    xT = jnp.transpose(x).reshape(-1)
    flats = []
    for l, t in enumerate(tables):
        f = t.reshape(-1)
        if _TSP[l] != f.shape[0]:
            f = jnp.pad(f, (0, _TSP[l] - f.shape[0]))
        flats.append(f)
    enc = _encode_sc(xT, flats)
    encT = enc.reshape(2 * _L, _N)
    out = _mlp_tc(encT, W1, b1, W2, b2)
    return out[0]


# speedup vs baseline: 3.3893x; 1.2333x over previous
import functools

import numpy as np
import jax
import jax.numpy as jnp
from jax import lax
from jax.experimental import pallas as pl
from jax.experimental.pallas import tpu as pltpu
from jax.experimental.pallas import tpu_sc as plsc

_L = 8
_FPL = 2
_T = 2 ** 19
_N_MIN = 32
_B_GROWTH = float(np.exp(np.log(2048.0 / _N_MIN) / (_L - 1)))
_RES = [int(np.floor(_N_MIN * (_B_GROWTH ** l))) for l in range(_L)]
_TS = [min(_T, (r + 1) ** 3) for r in _RES]
_DENSE = [(r + 1) ** 3 <= _T for r in _RES]
_K2 = -1640531535
_K3 = 805459861
_MASK = _T - 1
_TSP = [2 * s + ((-2 * s) % 128) for s in _TS]

_NC, _NS, _LANES = 2, 16, 16
_NW = _NC * _NS

_N = 262144
_PPW = _N // _NW
_C = 512
_NCHUNK = _PPW // _C
_NVEC = _C // _LANES

_CORNERS = [(dz, dy, dx) for dz in (0, 1) for dy in (0, 1) for dx in (0, 1)]


def _enc_body(xT, *rest):
    tabs = rest[:_L]
    enc = rest[_L]
    (xv0, xv1, xv2, wxs0, wys0, wzs0, wxs1, wys1, wzs1,
     idxE0, idxO0, idxE1, idxO1, bufE0, bufO0, bufE1, bufO1,
     outE, outO, sp0, sp1, semE0, semO0, semE1, semO1) = rest[_L + 1:]
    wxs = (wxs0, wxs1)
    wys = (wys0, wys1)
    wzs = (wzs0, wzs1)
    idxE = (idxE0, idxE1)
    idxO = (idxO0, idxO1)
    bufE = (bufE0, bufE1)
    bufO = (bufO0, bufO1)
    semE = (semE0, semE1)
    semO = (semO0, semO1)

    cid = lax.axis_index("c")
    sid = lax.axis_index("s")
    wid = sid * _NC + cid
    base = wid * _PPW

    pltpu.sync_copy(xT.at[pl.ds(0 * _N + base, _PPW)], xv0)
    pltpu.sync_copy(xT.at[pl.ds(1 * _N + base, _PPW)], xv1)
    pltpu.sync_copy(xT.at[pl.ds(2 * _N + base, _PPW)], xv2)

    @pl.when(sid == 0)
    def _stage():
        pltpu.sync_copy(tabs[0], sp0)
        pltpu.sync_copy(tabs[1], sp1)

    plsc.subcore_barrier()

    gather_src = [sp0, sp1] + list(tabs[2:])

    for l in range(_L):
        res = _RES[l]
        resf = float(res)

        def pass1(ch, b, l=l, res=res, resf=resf):
            def body(i, carry):
                off = pl.multiple_of(ch * _C + i * _LANES, _LANES)
                loc = pl.multiple_of(i * _LANES, _LANES)
                qs = []
                for xv, ws in ((xv0, wxs[b]), (xv1, wys[b]), (xv2, wzs[b])):
                    xd = xv[pl.ds(off, _LANES)]
                    xn = xd * 0.5 + 0.5
                    pos = xn * resf
                    q = pos.astype(jnp.int32)
                    w = pos - q.astype(jnp.float32)
                    ws[pl.ds(loc, _LANES)] = w
                    qs.append(q)
                q0, q1, q2 = qs
                if _DENSE[l]:
                    s = res + 1
                    eb = (q0 + q1 * s + q2 * (s * s)) * 2
                    for c, (dz, dy, dx) in enumerate(_CORNERS):
                        cst = 2 * (dx + dy * s + dz * s * s)
                        dst = pl.multiple_of(c * _C + i * _LANES, _LANES)
                        e = eb + cst
                        idxE[b][pl.ds(dst, _LANES)] = e
                        idxO[b][pl.ds(dst, _LANES)] = e + 1
                else:
                    ay0 = q1 * _K2
                    az0 = q2 * _K3
                    ay1 = ay0 + _K2
                    az1 = az0 + _K3
                    ax1 = q0 + 1
                    for c, (dz, dy, dx) in enumerate(_CORNERS):
                        hyz = (ay1 if dy else ay0) ^ (az1 if dz else az0)
                        h = (ax1 if dx else q0) ^ hyz
                        e = (h & _MASK) * 2
                        dst = pl.multiple_of(c * _C + i * _LANES, _LANES)
                        idxE[b][pl.ds(dst, _LANES)] = e
                        idxO[b][pl.ds(dst, _LANES)] = e + 1
                return carry

            lax.fori_loop(0, _NVEC, body, 0)

        src = gather_src[l]

        def issue(b, src=src):
            pltpu.make_async_copy(src.at[idxE[b]], bufE[b], semE[b]).start()
            pltpu.make_async_copy(src.at[idxO[b]], bufO[b], semO[b]).start()

        def drain(b, src=src):
            pltpu.make_async_copy(src.at[idxE[b]], bufE[b], semE[b]).wait()
            pltpu.make_async_copy(src.at[idxO[b]], bufO[b], semO[b]).wait()

        def pass2(ch, b):
            def body(i, carry):
                loc = pl.multiple_of(i * _LANES, _LANES)
                off = pl.multiple_of(ch * _C + i * _LANES, _LANES)
                wx = wxs[b][pl.ds(loc, _LANES)]
                wy = wys[b][pl.ds(loc, _LANES)]
                wz = wzs[b][pl.ds(loc, _LANES)]
                ux = 1.0 - wx
                uy = 1.0 - wy
                uz = 1.0 - wz
                tyz = {(0, 0): uy * uz, (0, 1): wy * uz,
                       (1, 0): uy * wz, (1, 1): wy * wz}
                acc0 = jnp.zeros((_LANES,), jnp.float32)
                acc1 = jnp.zeros((_LANES,), jnp.float32)
                for c, (dz, dy, dx) in enumerate(_CORNERS):
                    wfac = tyz[(dy, dz)] * (wx if dx else ux)
                    src = pl.multiple_of(c * _C + i * _LANES, _LANES)
                    acc0 = acc0 + wfac * bufE[b][pl.ds(src, _LANES)]
                    acc1 = acc1 + wfac * bufO[b][pl.ds(src, _LANES)]
                outE[pl.ds(off, _LANES)] = acc0
                outO[pl.ds(off, _LANES)] = acc1
                return carry

            lax.fori_loop(0, _NVEC, body, 0)

        pass1(0, 0)
        issue(0)
        pass1(1, 1)
        issue(1)

        def steady(g, carry):
            for b in (0, 1):
                chd = 2 * g + b
                drain(b)
                pass2(chd, b)
                pass1(chd + 2, b)
                issue(b)
            return carry

        lax.fori_loop(0, (_NCHUNK - 2) // 2, steady, 0)
        for b in (0, 1):
            drain(b)
            pass2(_NCHUNK - 2 + b, b)

        pltpu.sync_copy(outE, enc.at[pl.ds((2 * l) * _N + base, _PPW)])
        pltpu.sync_copy(outO, enc.at[pl.ds((2 * l + 1) * _N + base, _PPW)])


def _encode_sc(xT, flat_tables):
    mesh = plsc.VectorSubcoreMesh(core_axis_name="c", subcore_axis_name="s")
    scratch = (
        [pltpu.VMEM((_PPW,), jnp.float32)] * 3
        + [pltpu.VMEM((_C,), jnp.float32)] * 6
        + [pltpu.VMEM((8 * _C,), jnp.int32)] * 4
        + [pltpu.VMEM((8 * _C,), jnp.float32)] * 4
        + [pltpu.VMEM((_PPW,), jnp.float32)] * 2
        + [pltpu.VMEM_SHARED((_TSP[0],), jnp.float32)]
        + [pltpu.VMEM_SHARED((_TSP[1],), jnp.float32)]
        + [pltpu.SemaphoreType.DMA] * 4
    )
    fn = pl.kernel(
        _enc_body,
        out_type=jax.ShapeDtypeStruct((2 * _L * _N,), jnp.float32),
        mesh=mesh,
        scratch_types=scratch,
    )
    return fn(xT, *flat_tables)


def _mlp_body(enc_ref, w1_ref, b1_ref, w2_ref, b2_ref, o_ref):
    e = enc_ref[...]
    w1 = w1_ref[...]
    h = lax.dot_general(w1, e, (((0,), (0,)), ((), ())),
                        preferred_element_type=jnp.float32)
    h = jnp.maximum(h + b1_ref[...], 0.0)
    w2 = w2_ref[...]
    o = lax.dot_general(w2, h, (((0,), (0,)), ((), ())),
                        preferred_element_type=jnp.float32)
    o_ref[...] = jax.nn.sigmoid(o + b2_ref[...])


def _mlp_tc(encT, W1, b1, W2, b2):
    bn = 8192
    n = encT.shape[1]
    grid = (n // bn,)
    return pl.pallas_call(
        _mlp_body,
        grid=grid,
        in_specs=[
            pl.BlockSpec((2 * _L, bn), lambda j: (0, j)),
            pl.BlockSpec((2 * _L, 32), lambda j: (0, 0)),
            pl.BlockSpec((32, 1), lambda j: (0, 0)),
            pl.BlockSpec((32, 1), lambda j: (0, 0)),
            pl.BlockSpec((1, 1), lambda j: (0, 0)),
        ],
        out_specs=pl.BlockSpec((1, bn), lambda j: (0, j)),
        out_shape=jax.ShapeDtypeStruct((1, n), jnp.float32),
    )(encT, W1, b1.reshape(32, 1), W2, b2.reshape(1, 1))


def kernel(x, tables, W1, b1, W2, b2):
    xT = jnp.transpose(x).reshape(-1)
    flats = []
    for l, t in enumerate(tables):
        f = t.reshape(-1)
        if _TSP[l] != f.shape[0]:
            f = jnp.pad(f, (0, _TSP[l] - f.shape[0]))
        flats.append(f)
    enc = _encode_sc(xT, flats)
    encT = enc.reshape(2 * _L, _N)
    out = _mlp_tc(encT, W1, b1, W2, b2)
    return out[0]
